# Initial kernel scaffold; baseline (speedup 1.0000x reference)
import jax
import jax.numpy as jnp
from jax.experimental import pallas as pl


def kernel(x, W0, W1, W2, W3, W4, W5, W6, W7, W8):
    raise NotImplementedError("write your pallas kernel here")


# speedup vs baseline: 1.2710x; 1.2710x over previous
import functools

import jax
import jax.numpy as jnp
from jax import lax
from jax.experimental import pallas as pl
from jax.experimental.pallas import tpu as pltpu
from jax.experimental.pallas import tpu_sc as plsc

_N = 100000
_D = 128
_T_ROWS = 457
_NC = 2
_NS = 16
_NW = _NC * _NS
_OCT = _N // 8 // _NW
_OCT_EXTRA = _N // 8 - _OCT * _NW
_CHUNK = 16
_NCHUNKS = (_OCT + 1) * 8 // _CHUNK + 1


def _combine_tables(W0, W1, W2, W3, W4, W5, W6, W7, W8):
  g1 = (W1[:, None, :] + W4[None, :, :]).reshape(50, _D)
  g2 = (W2[:, None, :] + W3[None, :, :]).reshape(144, _D)
  g3 = (W5[:, None, :] + W6[None, :, :]).reshape(36, _D)
  g3 = (g3[:, None, :] + W7[None, :, :]).reshape(72, _D)
  g3 = (g3[:, None, :] + W8[None, :, :]).reshape(144, _D)
  return jnp.concatenate([W0, g1, g2, g3], axis=0).reshape(-1)


@functools.partial(
    pl.kernel,
    out_type=jax.ShapeDtypeStruct((_N * _D,), jnp.float32),
    mesh=plsc.VectorSubcoreMesh(core_axis_name="c", subcore_axis_name="s"),
    compiler_params=pltpu.CompilerParams(needs_layout_passes=False),
    scratch_types=[
        pltpu.VMEM((_T_ROWS * _D,), jnp.float32),
        pltpu.VMEM((_CHUNK * 9,), jnp.int32),
        pltpu.VMEM((_CHUNK * _D,), jnp.float32),
    ],
)
def _sc_lookup(x_hbm, t_hbm, out_hbm, t_v, x_v, o_v):
  wid = lax.axis_index("s") * _NC + lax.axis_index("c")
  w = wid.astype(jnp.int32)
  start8 = w * _OCT + jnp.minimum(w, _OCT_EXTRA)
  n8 = _OCT + (w < _OCT_EXTRA).astype(jnp.int32)
  wstart = start8 * 8
  wlast = wstart + n8 * 8 - _CHUNK

  pltpu.sync_copy(t_hbm, t_v)

  lane = lax.iota(jnp.int32, 16)
  xaddr = lane * 9
  obase = lane * _D

  def chunk(k, carry):
    base = jnp.minimum(wstart + k * _CHUNK, wlast)
    pltpu.sync_copy(x_hbm.at[pl.ds(base * 9, _CHUNK * 9)], x_v)
    xv = [plsc.load_gather(x_v, [xaddr + j]) for j in range(9)]
    i0 = xv[0]
    i1 = 119 + xv[1] * 10 + xv[4]
    i2 = 169 + xv[2] * 12 + xv[3]
    i3 = 313 + xv[5] * 24 + xv[6] * 4 + xv[7] * 2 + xv[8]
    a0 = i0 * _D
    a1 = i1 * _D
    a2 = i2 * _D
    a3 = i3 * _D
    ob = obase
    for c in range(_D):
      v = (plsc.load_gather(t_v, [a0]) + plsc.load_gather(t_v, [a1])) + (
          plsc.load_gather(t_v, [a2]) + plsc.load_gather(t_v, [a3]))
      plsc.store_scatter(o_v, [ob], v)
      if c != _D - 1:
        a0 = a0 + 1
        a1 = a1 + 1
        a2 = a2 + 1
        a3 = a3 + 1
        ob = ob + 1
    pltpu.sync_copy(o_v, out_hbm.at[pl.ds(base * _D, _CHUNK * _D)])
    return carry

  lax.fori_loop(0, _NCHUNKS, chunk, 0)


def kernel(x, W0, W1, W2, W3, W4, W5, W6, W7, W8):
  x32 = x.astype(jnp.int32).reshape(-1)
  t = _combine_tables(W0, W1, W2, W3, W4, W5, W6, W7, W8)
  out = _sc_lookup(x32, t)
  return out.reshape(_N, _D)


# speedup vs baseline: 1.3710x; 1.0787x over previous
import functools

import jax
import jax.numpy as jnp
from jax import lax
from jax.experimental import pallas as pl
from jax.experimental.pallas import tpu as pltpu
from jax.experimental.pallas import tpu_sc as plsc

_N = 100000
_D = 128
_T_ROWS = 457
_NC = 2
_NS = 16
_NW = _NC * _NS
_OCT = _N // 8 // _NW
_OCT_EXTRA = _N // 8 - _OCT * _NW
_CHUNK = 16
_NCHUNKS = (_OCT + 1) * 8 // _CHUNK + 1
_XROWS = 3136


def _combine_tables(W0, W1, W2, W3, W4, W5, W6, W7, W8):
  g1 = (W1[:, None, :] + W4[None, :, :]).reshape(50, _D)
  g2 = (W2[:, None, :] + W3[None, :, :]).reshape(144, _D)
  g3 = (W5[:, None, :] + W6[None, :, :]).reshape(36, _D)
  g3 = (g3[:, None, :] + W7[None, :, :]).reshape(72, _D)
  g3 = (g3[:, None, :] + W8[None, :, :]).reshape(144, _D)
  return jnp.concatenate([W0, g1, g2, g3], axis=0).reshape(-1)


@functools.partial(
    pl.kernel,
    out_type=jax.ShapeDtypeStruct((_N * _D,), jnp.float32),
    mesh=plsc.VectorSubcoreMesh(core_axis_name="c", subcore_axis_name="s"),
    compiler_params=pltpu.CompilerParams(needs_layout_passes=False),
    scratch_types=[
        pltpu.VMEM((_T_ROWS * _D,), jnp.float32),
        pltpu.VMEM((_XROWS * 9,), jnp.int32),
        pltpu.VMEM((_CHUNK * _D,), jnp.float32),
        pltpu.VMEM((_CHUNK * _D,), jnp.float32),
        pltpu.SemaphoreType.DMA,
        pltpu.SemaphoreType.DMA,
    ],
)
def _sc_lookup(x_hbm, t_hbm, out_hbm, t_v, x_v, o_a, o_b, sem_a, sem_b):
  wid = lax.axis_index("s") * _NC + lax.axis_index("c")
  w = wid.astype(jnp.int32)
  start8 = w * _OCT + jnp.minimum(w, _OCT_EXTRA)
  n8 = _OCT + (w < _OCT_EXTRA).astype(jnp.int32)
  wstart = start8 * 8
  wlast = wstart + n8 * 8 - _CHUNK
  xbase = jnp.minimum(wstart, _N - _XROWS)

  pltpu.sync_copy(x_hbm.at[pl.ds(xbase * 9, _XROWS * 9)], x_v)
  pltpu.sync_copy(t_hbm, t_v)

  lane = lax.iota(jnp.int32, 16)
  lane9 = lane * 9
  obase = lane * _D

  def do_chunk(i, o_ref):
    base = jnp.minimum(wstart + i * _CHUNK, wlast)
    xoff = (base - xbase) * 9
    xv = [plsc.load_gather(x_v, [xoff + lane9 + j]) for j in range(9)]
    i0 = xv[0]
    i1 = 119 + xv[1] * 10 + xv[4]
    i2 = 169 + xv[2] * 12 + xv[3]
    i3 = 313 + xv[5] * 24 + xv[6] * 4 + xv[7] * 2 + xv[8]
    a0 = i0 * _D
    a1 = i1 * _D
    a2 = i2 * _D
    a3 = i3 * _D
    ob = obase
    for c in range(_D):
      v = (plsc.load_gather(t_v, [a0]) + plsc.load_gather(t_v, [a1])) + (
          plsc.load_gather(t_v, [a2]) + plsc.load_gather(t_v, [a3]))
      plsc.store_scatter(o_ref, [ob], v)
      if c != _D - 1:
        a0 = a0 + 1
        a1 = a1 + 1
        a2 = a2 + 1
        a3 = a3 + 1
        ob = ob + 1
    return base

  def body(k2, carry):
    @pl.when(k2 > 0)
    def _():
      pltpu.make_async_copy(
          o_a, out_hbm.at[pl.ds(0, _CHUNK * _D)], sem_a).wait()
    base_a = do_chunk(k2 * 2, o_a)
    pltpu.async_copy(o_a, out_hbm.at[pl.ds(base_a * _D, _CHUNK * _D)], sem_a)

    @pl.when(k2 > 0)
    def _():
      pltpu.make_async_copy(
          o_b, out_hbm.at[pl.ds(0, _CHUNK * _D)], sem_b).wait()
    base_b = do_chunk(k2 * 2 + 1, o_b)
    pltpu.async_copy(o_b, out_hbm.at[pl.ds(base_b * _D, _CHUNK * _D)], sem_b)
    return carry

  lax.fori_loop(0, _NCHUNKS // 2, body, 0)
  pltpu.make_async_copy(o_a, out_hbm.at[pl.ds(0, _CHUNK * _D)], sem_a).wait()
  pltpu.make_async_copy(o_b, out_hbm.at[pl.ds(0, _CHUNK * _D)], sem_b).wait()


def kernel(x, W0, W1, W2, W3, W4, W5, W6, W7, W8):
  x32 = x.astype(jnp.int32).reshape(-1)
  t = _combine_tables(W0, W1, W2, W3, W4, W5, W6, W7, W8)
  out = _sc_lookup(x32, t)
  return out.reshape(_N, _D)


# speedup vs baseline: 6.2163x; 4.5342x over previous
import functools

import jax
import jax.numpy as jnp
from jax import lax
from jax.experimental import pallas as pl
from jax.experimental.pallas import tpu as pltpu
from jax.experimental.pallas import tpu_sc as plsc

_N = 100000
_D = 128
_T_ROWS = 457
_NC = 2
_NS = 16
_NW = _NC * _NS
_OCT = _N // 8 // _NW
_OCT_EXTRA = _N // 8 - _OCT * _NW
_CHUNK = 32
_NPAIRS = 49
_XROWS = 3136

_BCAST_DN = lax.GatherDimensionNumbers(
    offset_dims=(), collapsed_slice_dims=(0,), start_index_map=(0,))


def _bcast(vec, r):
  return lax.gather(vec, jnp.full((16, 1), r, jnp.int32), _BCAST_DN, (1,),
                    mode=lax.GatherScatterMode.PROMISE_IN_BOUNDS)


def _combine_tables(W0, W1, W2, W3, W4, W5, W6, W7, W8):
  g1 = (W1[:, None, :] + W4[None, :, :]).reshape(50, _D)
  g2 = (W2[:, None, :] + W3[None, :, :]).reshape(144, _D)
  g3 = (W5[:, None, :] + W6[None, :, :]).reshape(36, _D)
  g3 = (g3[:, None, :] + W7[None, :, :]).reshape(72, _D)
  g3 = (g3[:, None, :] + W8[None, :, :]).reshape(144, _D)
  return jnp.concatenate([W0, g1, g2, g3], axis=0).reshape(-1)


@functools.partial(
    pl.kernel,
    out_type=jax.ShapeDtypeStruct((_N * _D,), jnp.float32),
    mesh=plsc.VectorSubcoreMesh(core_axis_name="c", subcore_axis_name="s"),
    compiler_params=pltpu.CompilerParams(needs_layout_passes=False),
    scratch_types=[
        pltpu.VMEM((_T_ROWS * _D,), jnp.float32),
        pltpu.VMEM((_XROWS * 9,), jnp.int32),
        pltpu.VMEM((_CHUNK * _D,), jnp.float32),
        pltpu.VMEM((_CHUNK * _D,), jnp.float32),
        pltpu.SemaphoreType.DMA,
        pltpu.SemaphoreType.DMA,
    ],
)
def _sc_lookup(x_hbm, t_hbm, out_hbm, t_v, x_v, o_a, o_b, sem_a, sem_b):
  wid = lax.axis_index("s") * _NC + lax.axis_index("c")
  w = wid.astype(jnp.int32)
  start8 = w * _OCT + jnp.minimum(w, _OCT_EXTRA)
  n8 = _OCT + (w < _OCT_EXTRA).astype(jnp.int32)
  wstart = start8 * 8
  wlast = wstart + n8 * 8 - _CHUNK
  xbase = jnp.minimum(wstart, _N - _XROWS)

  pltpu.sync_copy(x_hbm.at[pl.ds(xbase * 9, _XROWS * 9)], x_v)
  pltpu.sync_copy(t_hbm, t_v)

  lane = lax.iota(jnp.int32, 16)
  lane9 = lane * 9
  coffs = [lane + 16 * c8 for c8 in range(_D // 16)]

  def do_subchunk(base, r0, o_ref):
    xoff = (base + r0 - xbase) * 9
    xv = [plsc.load_gather(x_v, [xoff + lane9 + j]) for j in range(9)]
    i0 = xv[0]
    i1 = 119 + xv[1] * 10 + xv[4]
    i2 = 169 + xv[2] * 12 + xv[3]
    i3 = 313 + xv[5] * 24 + xv[6] * 4 + xv[7] * 2 + xv[8]
    a0 = i0 * _D
    a1 = i1 * _D
    a2 = i2 * _D
    a3 = i3 * _D
    for r in range(16):
      b0 = _bcast(a0, r)
      b1 = _bcast(a1, r)
      b2 = _bcast(a2, r)
      b3 = _bcast(a3, r)
      quads = []
      for co in coffs:
        quads.append((plsc.load_gather(t_v, [b0 + co]),
                      plsc.load_gather(t_v, [b1 + co]),
                      plsc.load_gather(t_v, [b2 + co]),
                      plsc.load_gather(t_v, [b3 + co])))
      off = (r0 + r) * _D
      for c8 in range(_D // 16):
        q = quads[c8]
        o_ref[pl.ds(off + c8 * 16, 16)] = (q[0] + q[1]) + (q[2] + q[3])

  def do_chunk(i, o_ref):
    base = jnp.minimum(wstart + i * _CHUNK, wlast)
    for r0 in range(0, _CHUNK, 16):
      do_subchunk(base, r0, o_ref)
    return base

  def body(k2, carry):
    @pl.when(k2 > 0)
    def _():
      pltpu.make_async_copy(
          o_a, out_hbm.at[pl.ds(0, _CHUNK * _D)], sem_a).wait()
    base_a = do_chunk(k2 * 2, o_a)
    pltpu.async_copy(o_a, out_hbm.at[pl.ds(base_a * _D, _CHUNK * _D)], sem_a)

    @pl.when(k2 > 0)
    def _():
      pltpu.make_async_copy(
          o_b, out_hbm.at[pl.ds(0, _CHUNK * _D)], sem_b).wait()
    base_b = do_chunk(k2 * 2 + 1, o_b)
    pltpu.async_copy(o_b, out_hbm.at[pl.ds(base_b * _D, _CHUNK * _D)], sem_b)
    return carry

  lax.fori_loop(0, _NPAIRS, body, 0)
  pltpu.make_async_copy(o_a, out_hbm.at[pl.ds(0, _CHUNK * _D)], sem_a).wait()
  pltpu.make_async_copy(o_b, out_hbm.at[pl.ds(0, _CHUNK * _D)], sem_b).wait()


def kernel(x, W0, W1, W2, W3, W4, W5, W6, W7, W8):
  x32 = x.astype(jnp.int32).reshape(-1)
  t = _combine_tables(W0, W1, W2, W3, W4, W5, W6, W7, W8)
  out = _sc_lookup(x32, t)
  return out.reshape(_N, _D)


# speedup vs baseline: 10.6673x; 1.7160x over previous
import functools

import jax
import jax.numpy as jnp
from jax import lax
from jax.experimental import pallas as pl
from jax.experimental.pallas import tpu as pltpu
from jax.experimental.pallas import tpu_sc as plsc

_N = 100000
_D = 128
_T_ROWS = 457
_NC = 2
_NS = 16
_NW = _NC * _NS
_OCT = _N // 8 // _NW
_OCT_EXTRA = _N // 8 - _OCT * _NW
_CHUNK = 32
_NPAIRS = 49
_XROWS = 3136

_BCAST_DN = lax.GatherDimensionNumbers(
    offset_dims=(), collapsed_slice_dims=(0,), start_index_map=(0,))


def _bcast(vec, r):
  return lax.gather(vec, jnp.full((16, 1), r, jnp.int32), _BCAST_DN, (1,),
                    mode=lax.GatherScatterMode.PROMISE_IN_BOUNDS)


def _combine_tables(W0, W1, W2, W3, W4, W5, W6, W7, W8):
  g1 = (W1[:, None, :] + W4[None, :, :]).reshape(50, _D)
  g2 = (W2[:, None, :] + W3[None, :, :]).reshape(144, _D)
  g3 = (W5[:, None, :] + W6[None, :, :]).reshape(36, _D)
  g3 = (g3[:, None, :] + W7[None, :, :]).reshape(72, _D)
  g3 = (g3[:, None, :] + W8[None, :, :]).reshape(144, _D)
  return jnp.concatenate([W0, g1, g2, g3], axis=0).reshape(-1)


@functools.partial(
    pl.kernel,
    out_type=jax.ShapeDtypeStruct((_N * _D,), jnp.float32),
    mesh=plsc.VectorSubcoreMesh(core_axis_name="c", subcore_axis_name="s"),
    compiler_params=pltpu.CompilerParams(needs_layout_passes=False),
    scratch_types=[
        pltpu.VMEM((_T_ROWS * _D,), jnp.float32),
        pltpu.VMEM((_XROWS * 9,), jnp.int32),
        pltpu.VMEM((_CHUNK * _D,), jnp.float32),
        pltpu.VMEM((_CHUNK * _D,), jnp.float32),
        pltpu.SemaphoreType.DMA,
        pltpu.SemaphoreType.DMA,
    ],
)
def _sc_lookup(x_hbm, t_hbm, out_hbm, t_v, x_v, o_a, o_b, sem_a, sem_b):
  wid = lax.axis_index("s") * _NC + lax.axis_index("c")
  w = wid.astype(jnp.int32)
  start8 = w * _OCT + jnp.minimum(w, _OCT_EXTRA)
  n8 = _OCT + (w < _OCT_EXTRA).astype(jnp.int32)
  wstart = start8 * 8
  wlast = wstart + n8 * 8 - _CHUNK
  xbase = jnp.minimum(wstart, _N - _XROWS)

  pltpu.sync_copy(x_hbm.at[pl.ds(xbase * 9, _XROWS * 9)], x_v)
  pltpu.sync_copy(t_hbm, t_v)

  lane = lax.iota(jnp.int32, 16)
  lane9 = lane * 9
  coffs = [lane + 16 * c8 for c8 in range(_D // 16)]

  def do_subchunk(base, r0, o_ref):
    xoff = (base + r0 - xbase) * 9
    xv = [plsc.load_gather(x_v, [xoff + lane9 + j]) for j in range(9)]
    i0 = xv[0]
    i1 = 119 + xv[1] * 10 + xv[4]
    i2 = 169 + xv[2] * 12 + xv[3]
    i3 = 313 + xv[5] * 24 + xv[6] * 4 + xv[7] * 2 + xv[8]
    a0 = i0 * _D
    a1 = i1 * _D
    a2 = i2 * _D
    a3 = i3 * _D

    def row_body(r, carry):
      ridx = jnp.full((16, 1), r, jnp.int32)
      b0 = lax.gather(a0, ridx, _BCAST_DN, (1,),
                      mode=lax.GatherScatterMode.PROMISE_IN_BOUNDS)
      b1 = lax.gather(a1, ridx, _BCAST_DN, (1,),
                      mode=lax.GatherScatterMode.PROMISE_IN_BOUNDS)
      b2 = lax.gather(a2, ridx, _BCAST_DN, (1,),
                      mode=lax.GatherScatterMode.PROMISE_IN_BOUNDS)
      b3 = lax.gather(a3, ridx, _BCAST_DN, (1,),
                      mode=lax.GatherScatterMode.PROMISE_IN_BOUNDS)
      quads = []
      for co in coffs:
        quads.append((plsc.load_gather(t_v, [b0 + co]),
                      plsc.load_gather(t_v, [b1 + co]),
                      plsc.load_gather(t_v, [b2 + co]),
                      plsc.load_gather(t_v, [b3 + co])))
      off = (r0 + r) * _D
      for c8 in range(_D // 16):
        q = quads[c8]
        o_ref[pl.ds(off + c8 * 16, 16)] = (q[0] + q[1]) + (q[2] + q[3])
      return carry

    lax.fori_loop(0, 16, row_body, 0)

  def do_chunk(i, o_ref):
    base = jnp.minimum(wstart + i * _CHUNK, wlast)
    for r0 in range(0, _CHUNK, 16):
      do_subchunk(base, r0, o_ref)
    return base

  def body(k2, carry):
    @pl.when(k2 > 0)
    def _():
      pltpu.make_async_copy(
          o_a, out_hbm.at[pl.ds(0, _CHUNK * _D)], sem_a).wait()
    base_a = do_chunk(k2 * 2, o_a)
    pltpu.async_copy(o_a, out_hbm.at[pl.ds(base_a * _D, _CHUNK * _D)], sem_a)

    @pl.when(k2 > 0)
    def _():
      pltpu.make_async_copy(
          o_b, out_hbm.at[pl.ds(0, _CHUNK * _D)], sem_b).wait()
    base_b = do_chunk(k2 * 2 + 1, o_b)
    pltpu.async_copy(o_b, out_hbm.at[pl.ds(base_b * _D, _CHUNK * _D)], sem_b)
    return carry

  lax.fori_loop(0, _NPAIRS, body, 0)
  pltpu.make_async_copy(o_a, out_hbm.at[pl.ds(0, _CHUNK * _D)], sem_a).wait()
  pltpu.make_async_copy(o_b, out_hbm.at[pl.ds(0, _CHUNK * _D)], sem_b).wait()


def kernel(x, W0, W1, W2, W3, W4, W5, W6, W7, W8):
  x32 = x.astype(jnp.int32).reshape(-1)
  t = _combine_tables(W0, W1, W2, W3, W4, W5, W6, W7, W8)
  out = _sc_lookup(x32, t)
  return out.reshape(_N, _D)
